# baseline (speedup 1.0000x reference)
import functools

import jax
import jax.numpy as jnp
from jax import lax
from jax.experimental import pallas as pl
from jax.experimental.pallas import tpu as pltpu
from jax.experimental.pallas import tpu_sc as plsc

VOCAB = 13
EMBED_DIM = 1024
BATCH = 4
SEQ = 8192

_B = BATCH * SEQ
_NW = 32
_BPW = _B // _NW
_C = 32
_NCH = _BPW // _C
_VPAD = 16
_L = 16

_mesh = plsc.VectorSubcoreMesh(core_axis_name="c", subcore_axis_name="s")


@functools.partial(
    pl.kernel,
    mesh=_mesh,
    out_type=jax.ShapeDtypeStruct((_B, EMBED_DIM), jnp.float32),
    scratch_types=[
        pltpu.VMEM((_BPW,), jnp.int32),
        pltpu.VMEM((_C, EMBED_DIM), jnp.float32),
        pltpu.VMEM((_C, EMBED_DIM), jnp.float32),
        pltpu.VMEM((_C, EMBED_DIM), jnp.float32),
        pltpu.SemaphoreType.DMA,
        pltpu.SemaphoreType.DMA,
        pltpu.SemaphoreType.DMA,
        pltpu.SemaphoreType.DMA,
        pltpu.SemaphoreType.DMA,
        pltpu.SemaphoreType.DMA,
    ],
    compiler_params=pltpu.CompilerParams(needs_layout_passes=False),
)
def _emb(x_hbm, table_hbm, out_hbm, idx_v, rows0, rows1, rows2,
         gsem0, gsem1, gsem2, ssem0, ssem1, ssem2):
    wid = lax.axis_index("s") * 2 + lax.axis_index("c")
    base = wid * _BPW

    pltpu.sync_copy(x_hbm.at[pl.ds(base, _BPW)], idx_v)

    woff = jnp.full((_L,), wid * _VPAD, jnp.int32)
    for k in range(_BPW // _L):
        sl = pl.ds(k * _L, _L)
        idx_v[sl] = idx_v[sl] + woff

    rows = (rows0, rows1, rows2)
    gsem = (gsem0, gsem1, gsem2)
    ssem = (ssem0, ssem1, ssem2)

    def g_start(b, j):
        pltpu.async_copy(table_hbm.at[idx_v.at[pl.ds(j * _C, _C)]],
                         rows[b], gsem[b])

    def g_wait(b, j):
        pltpu.make_async_copy(table_hbm.at[idx_v.at[pl.ds(j * _C, _C)]],
                              rows[b], gsem[b]).wait()

    def s_start(b, j):
        pltpu.async_copy(rows[b], out_hbm.at[pl.ds(base + j * _C, _C)],
                         ssem[b])

    def s_wait(b, j):
        pltpu.make_async_copy(rows[b], out_hbm.at[pl.ds(base + j * _C, _C)],
                              ssem[b]).wait()

    def step(b, j):
        g_wait(b, j)
        s_start(b, j)
        b2 = (b + 2) % 3
        s_wait(b2, j - 1)
        g_start(b2, j + 2)

    g_start(0, 0)
    g_start(1, 1)
    g_wait(0, 0)
    s_start(0, 0)
    g_start(2, 2)

    def body(t, carry):
        for db in (1, 2, 3):
            step(db % 3, 3 * t + db)
        return carry

    lax.fori_loop(0, (_NCH - 5) // 3, body, 0)

    step(1, _NCH - 4)
    step(2, _NCH - 3)
    g_wait(0, _NCH - 2)
    s_start(0, _NCH - 2)
    g_wait(1, _NCH - 1)
    s_start(1, _NCH - 1)
    s_wait(2, _NCH - 3)
    s_wait(0, _NCH - 2)
    s_wait(1, _NCH - 1)


def kernel(x, table):
    table_padded = jnp.pad(table, ((0, _VPAD - VOCAB), (0, 0)))
    table_rep = jnp.tile(table_padded, (_NW, 1))
    out = _emb(x.reshape(_B).astype(jnp.int32), table_rep)
    return out.reshape(BATCH, SEQ, EMBED_DIM)
